# baseline (speedup 1.0000x reference)
import functools

import jax
import jax.numpy as jnp
from jax import lax
from jax.experimental import pallas as pl
from jax.experimental.pallas import tpu as pltpu
from jax.experimental.pallas import tpu_sc as plsc

N_NODES = 10000
D = 128
N_LAYERS = 5
E = 320000

NUM_CORES = 2
NUM_SUBCORES = 16
NUM_WORKERS = NUM_CORES * NUM_SUBCORES

CH = 128
NCH = 80
PBLK = 8
E_PAD = NUM_WORKERS * NCH * CH
ROWS_PER_SUB = 640
N_SP = NUM_SUBCORES * ROWS_PER_SUB

ROW_BLK = 1000

_mesh = plsc.VectorSubcoreMesh(core_axis_name="c", subcore_axis_name="s")


@functools.partial(
    pl.kernel,
    mesh=_mesh,
    out_type=jax.ShapeDtypeStruct((NUM_CORES, N_SP, D), jnp.float32),
    scratch_types=[
        pltpu.VMEM((PBLK * CH,), jnp.int32),
        pltpu.VMEM((CH,), jnp.int32),
        pltpu.VMEM((CH,), jnp.int32),
        pltpu.VMEM((CH, D), jnp.float32),
        pltpu.VMEM_SHARED((N_SP, D), jnp.float32),
        pltpu.SemaphoreType.DMA,
    ],
)
def _segment_sum_sc(h_hbm, pk_hbm, zeros_hbm, out_hbm,
                    pair_v, src_v, dst_v, rows_v, acc_sh, sem):
    cid = lax.axis_index("c")
    sid = lax.axis_index("s")
    wid = cid * NUM_SUBCORES + sid

    pltpu.sync_copy(zeros_hbm, acc_sh.at[pl.ds(sid * ROWS_PER_SUB, ROWS_PER_SUB)])
    plsc.subcore_barrier()

    base = wid * (NCH * CH)

    @pl.loop(0, NCH // PBLK)
    def _(j):
        pltpu.sync_copy(pk_hbm.at[pl.ds(base + j * (PBLK * CH), PBLK * CH)], pair_v)
        for c in range(PBLK):
            for k in range(CH // 16):
                v = pair_v[pl.ds(c * CH + k * 16, 16)]
                src_v[pl.ds(k * 16, 16)] = jnp.bitwise_and(v, 0xFFFF)
                dst_v[pl.ds(k * 16, 16)] = jnp.right_shift(v, 16)
            pltpu.async_copy(h_hbm.at[src_v], rows_v, sem).wait()
            pltpu.sync_copy(rows_v, acc_sh.at[dst_v], add=True)

    plsc.subcore_barrier()
    pltpu.sync_copy(
        acc_sh.at[pl.ds(sid * ROWS_PER_SUB, ROWS_PER_SUB)],
        out_hbm.at[cid, pl.ds(sid * ROWS_PER_SUB, ROWS_PER_SUB)],
    )


def _encoder_body(x_ref, w_ref, b_ref, o_ref):
    acc = jnp.dot(x_ref[...], w_ref[...], preferred_element_type=jnp.float32)
    o_ref[...] = jnp.maximum(acc + b_ref[...], 0.0)


def _encoder(x, W, b):
    return pl.pallas_call(
        _encoder_body,
        grid=(N_NODES // ROW_BLK,),
        in_specs=[
            pl.BlockSpec((ROW_BLK, D), lambda i: (i, 0)),
            pl.BlockSpec((D, D), lambda i: (0, 0)),
            pl.BlockSpec((1, D), lambda i: (0, 0)),
        ],
        out_specs=pl.BlockSpec((ROW_BLK, D), lambda i: (i, 0)),
        out_shape=jax.ShapeDtypeStruct((N_NODES, D), jnp.float32),
    )(x, W, b.reshape(1, D))


def _gin_mlp_body(relu_out, h_ref, p_ref, s_ref, w1_ref, b1_ref, w2_ref, b2_ref, o_ref):
    z = s_ref[0, 0] * h_ref[...] + p_ref[0] + p_ref[1]
    t = jnp.dot(z, w1_ref[...], preferred_element_type=jnp.float32) + b1_ref[...]
    t = jnp.maximum(t, 0.0)
    o = jnp.dot(t, w2_ref[...], preferred_element_type=jnp.float32) + b2_ref[...]
    if relu_out:
        o = jnp.maximum(o, 0.0)
    o_ref[...] = o


def _gin_mlp(relu_out, h, parts, scale, W1l, b1l, W2l, b2l):
    return pl.pallas_call(
        functools.partial(_gin_mlp_body, relu_out),
        grid=(N_NODES // ROW_BLK,),
        in_specs=[
            pl.BlockSpec((ROW_BLK, D), lambda i: (i, 0)),
            pl.BlockSpec((NUM_CORES, ROW_BLK, D), lambda i: (0, i, 0)),
            pl.BlockSpec((1, 1), lambda i: (0, 0)),
            pl.BlockSpec((D, D), lambda i: (0, 0)),
            pl.BlockSpec((1, D), lambda i: (0, 0)),
            pl.BlockSpec((D, D), lambda i: (0, 0)),
            pl.BlockSpec((1, D), lambda i: (0, 0)),
        ],
        out_specs=pl.BlockSpec((ROW_BLK, D), lambda i: (i, 0)),
        out_shape=jax.ShapeDtypeStruct((N_NODES, D), jnp.float32),
    )(h, parts, scale, W1l, b1l.reshape(1, D), W2l, b2l.reshape(1, D))


def kernel(x, edge_index, W_enc, b_enc, eps, W1, b1, W2, b2):
    src = edge_index[0]
    dst = edge_index[1]
    pad = E_PAD - E
    src_p = jnp.concatenate([src, jnp.zeros((pad,), jnp.int32)])
    dst_p = jnp.concatenate([dst, jnp.full((pad,), N_NODES, jnp.int32)])
    packed = jnp.bitwise_or(src_p, jnp.left_shift(dst_p, 16))
    zeros_blk = jnp.zeros((ROWS_PER_SUB, D), jnp.float32)

    h = _encoder(x, W_enc, b_enc)
    for l in range(N_LAYERS):
        parts = _segment_sum_sc(h, packed, zeros_blk)
        scale = (1.0 + eps[l]).reshape(1, 1)
        h = _gin_mlp(l < N_LAYERS - 1, h, parts, scale, W1[l], b1[l], W2[l], b2[l])
    return h


# speedup vs baseline: 1.4585x; 1.4585x over previous
import functools

import jax
import jax.numpy as jnp
from jax import lax
from jax.experimental import pallas as pl
from jax.experimental.pallas import tpu as pltpu
from jax.experimental.pallas import tpu_sc as plsc

N_NODES = 10000
D = 128
N_LAYERS = 5
E = 320000

NUM_CORES = 2
NUM_SUBCORES = 16
NUM_WORKERS = NUM_CORES * NUM_SUBCORES

CH = 128
NCH = 79
E_PAD = NUM_WORKERS * NCH * CH
ROWS_PER_SUB = 640
N_SP = NUM_SUBCORES * ROWS_PER_SUB

ROW_BLK = 1000

_mesh = plsc.VectorSubcoreMesh(core_axis_name="c", subcore_axis_name="s")


@functools.partial(
    pl.kernel,
    mesh=_mesh,
    out_type=jax.ShapeDtypeStruct((NUM_CORES, N_SP, D), jnp.float32),
    scratch_types=[
        pltpu.VMEM((CH,), jnp.int32),
        pltpu.VMEM((CH,), jnp.int32),
        pltpu.VMEM((CH,), jnp.int32),
        pltpu.VMEM((CH, D), jnp.float32),
        pltpu.VMEM_SHARED((N_SP, D), jnp.float32),
        pltpu.SemaphoreType.DMA,
    ],
)
def _segment_sum_sc(h_hbm, pk_hbm, zeros_hbm, out_hbm,
                    pair_v, src_v, dst_v, rows_v, acc_sh, sem):
    cid = lax.axis_index("c")
    sid = lax.axis_index("s")
    wid = cid * NUM_SUBCORES + sid

    pltpu.sync_copy(zeros_hbm, acc_sh.at[pl.ds(sid * ROWS_PER_SUB, ROWS_PER_SUB)])
    plsc.subcore_barrier()

    base = wid * (NCH * CH)

    @pl.loop(0, NCH)
    def _(i):
        pltpu.sync_copy(pk_hbm.at[pl.ds(base + i * CH, CH)], pair_v)
        for k in range(CH // 16):
            v = pair_v[pl.ds(k * 16, 16)]
            src_v[pl.ds(k * 16, 16)] = jnp.bitwise_and(v, 0xFFFF)
            dst_v[pl.ds(k * 16, 16)] = jnp.right_shift(v, 16)
        pltpu.async_copy(h_hbm.at[src_v], rows_v, sem).wait()
        pltpu.sync_copy(rows_v, acc_sh.at[dst_v], add=True)

    plsc.subcore_barrier()
    pltpu.sync_copy(
        acc_sh.at[pl.ds(sid * ROWS_PER_SUB, ROWS_PER_SUB)],
        out_hbm.at[cid, pl.ds(sid * ROWS_PER_SUB, ROWS_PER_SUB)],
    )


def _encoder_body(x_ref, w_ref, b_ref, o_ref):
    acc = jnp.dot(x_ref[...], w_ref[...], preferred_element_type=jnp.float32)
    o_ref[...] = jnp.maximum(acc + b_ref[...], 0.0)


def _encoder(x, W, b):
    return pl.pallas_call(
        _encoder_body,
        grid=(N_NODES // ROW_BLK,),
        in_specs=[
            pl.BlockSpec((ROW_BLK, D), lambda i: (i, 0)),
            pl.BlockSpec((D, D), lambda i: (0, 0)),
            pl.BlockSpec((1, D), lambda i: (0, 0)),
        ],
        out_specs=pl.BlockSpec((ROW_BLK, D), lambda i: (i, 0)),
        out_shape=jax.ShapeDtypeStruct((N_NODES, D), jnp.float32),
    )(x, W, b.reshape(1, D))


def _gin_mlp_body(relu_out, h_ref, p_ref, s_ref, w1_ref, b1_ref, w2_ref, b2_ref, o_ref):
    z = s_ref[0, 0] * h_ref[...] + p_ref[0] + p_ref[1]
    t = jnp.dot(z, w1_ref[...], preferred_element_type=jnp.float32) + b1_ref[...]
    t = jnp.maximum(t, 0.0)
    o = jnp.dot(t, w2_ref[...], preferred_element_type=jnp.float32) + b2_ref[...]
    if relu_out:
        o = jnp.maximum(o, 0.0)
    o_ref[...] = o


def _gin_mlp(relu_out, h, parts, scale, W1l, b1l, W2l, b2l):
    return pl.pallas_call(
        functools.partial(_gin_mlp_body, relu_out),
        grid=(N_NODES // ROW_BLK,),
        in_specs=[
            pl.BlockSpec((ROW_BLK, D), lambda i: (i, 0)),
            pl.BlockSpec((NUM_CORES, ROW_BLK, D), lambda i: (0, i, 0)),
            pl.BlockSpec((1, 1), lambda i: (0, 0)),
            pl.BlockSpec((D, D), lambda i: (0, 0)),
            pl.BlockSpec((1, D), lambda i: (0, 0)),
            pl.BlockSpec((D, D), lambda i: (0, 0)),
            pl.BlockSpec((1, D), lambda i: (0, 0)),
        ],
        out_specs=pl.BlockSpec((ROW_BLK, D), lambda i: (i, 0)),
        out_shape=jax.ShapeDtypeStruct((N_NODES, D), jnp.float32),
    )(h, parts, scale, W1l, b1l.reshape(1, D), W2l, b2l.reshape(1, D))


def kernel(x, edge_index, W_enc, b_enc, eps, W1, b1, W2, b2):
    src = edge_index[0]
    dst = edge_index[1]
    pad = E_PAD - E
    src_p = jnp.concatenate([src, jnp.zeros((pad,), jnp.int32)])
    dst_p = jnp.concatenate([dst, jnp.full((pad,), N_NODES, jnp.int32)])
    packed = jnp.bitwise_or(src_p, jnp.left_shift(dst_p, 16))
    zeros_blk = jnp.zeros((ROWS_PER_SUB, D), jnp.float32)

    h = _encoder(x, W_enc, b_enc)
    for l in range(N_LAYERS):
        parts = _segment_sum_sc(h, packed, zeros_blk)
        scale = (1.0 + eps[l]).reshape(1, 1)
        h = _gin_mlp(l < N_LAYERS - 1, h, parts, scale, W1[l], b1[l], W2[l], b2[l])
    return h
